# baseline (speedup 1.0000x reference)
import functools

import jax
import jax.numpy as jnp
from jax import lax
from jax.experimental import pallas as pl
from jax.experimental.pallas import tpu as pltpu
from jax.experimental.pallas import tpu_sc as plsc

B = 16384
D = 64
NNEG = 20
NV = NNEG + 1
L = 16
NC = 2
NS = 16
NW = NC * NS
BW = B // NW
CB = 32
NCHUNK = BW // CB
GJ = 6
GN = CB * NV // GJ
W128 = 2 * D


def _sc_body(upos_hbm, vidx_hbm, uw_hbm, vw_hbm, pos_hbm, neg_hbm,
             uidx_v, urow_v, vidx_v, vrow_v, urows, vrows, posb, negb, sem):
    wid = lax.axis_index("s") * NC + lax.axis_index("c")

    def chunk_body(c, carry):
        gbase = wid * BW + c * CB

        pltpu.sync_copy(upos_hbm.at[pl.ds(gbase, CB)], uidx_v.at[pl.ds(0, CB)])
        pltpu.sync_copy(vidx_hbm.at[pl.ds(gbase * NV, CB * NV)],
                        vidx_v.at[pl.ds(0, CB * NV)])

        for i in range(CB // L):
            urow_v[pl.ds(i * L, L)] = lax.shift_right_logical(
                uidx_v[pl.ds(i * L, L)], 1)
        for i in range(CB * NV // L):
            vrow_v[pl.ds(i * L, L)] = lax.shift_right_logical(
                vidx_v[pl.ds(i * L, L)], 1)

        copies = [pltpu.async_copy(uw_hbm.at[urow_v], urows, sem)]
        for j in range(GJ):
            copies.append(pltpu.async_copy(
                vw_hbm.at[vrow_v.at[pl.ds(j * GN, GN)]],
                vrows.at[pl.ds(j * GN, GN)], sem))
        for cp in copies:
            cp.wait()

        def bbody(b, carry2):
            rb = b * NV
            upar = uidx_v[pl.ds(b, L)]
            uoff = (upar[0] & 1) * D
            u = [urows[b, pl.ds(uoff + 16 * k, 16)] for k in range(4)]
            vpar = vidx_v[pl.ds(rb, L)]
            voff = (vpar[0] & 1) * D
            v = [vrows[rb, pl.ds(voff + 16 * k, 16)] for k in range(4)]
            aoff = (vpar[1] & 1) * D
            acc = [vrows[rb + 1, pl.ds(aoff + 16 * k, 16)] for k in range(4)]
            vpar2 = vidx_v[pl.ds(rb + L, L)]
            for n in range(2, NV):
                pe = vpar[n] if n < L else vpar2[n - L]
                noff = (pe & 1) * D
                for k in range(4):
                    acc[k] = acc[k] + vrows[rb + n, pl.ds(noff + 16 * k, 16)]
            pos = u[0] * v[0] + u[1] * v[1] + u[2] * v[2] + u[3] * v[3]
            neg = u[0] * acc[0] + u[1] * acc[1] + u[2] * acc[2] + u[3] * acc[3]
            posb[pl.ds(b * L, L)] = pos
            negb[pl.ds(b * L, L)] = neg
            return carry2

        lax.fori_loop(0, CB, bbody, 0, unroll=False)

        pltpu.sync_copy(posb, pos_hbm.at[pl.ds(gbase * L, CB * L)])
        pltpu.sync_copy(negb, neg_hbm.at[pl.ds(gbase * L, CB * L)])
        return carry

    lax.fori_loop(0, NCHUNK, chunk_body, 0, unroll=False)


_sc_call = functools.partial(
    pl.kernel,
    out_type=(jax.ShapeDtypeStruct((B * L,), jnp.float32),
              jax.ShapeDtypeStruct((B * L,), jnp.float32)),
    mesh=plsc.VectorSubcoreMesh(core_axis_name="c", subcore_axis_name="s"),
    compiler_params=pltpu.CompilerParams(use_tc_tiling_on_sc=True),
    scratch_types=[
        pltpu.VMEM((CB + L,), jnp.int32),
        pltpu.VMEM((CB,), jnp.int32),
        pltpu.VMEM((CB * NV + 2 * L,), jnp.int32),
        pltpu.VMEM((CB * NV,), jnp.int32),
        pltpu.VMEM((CB, W128), jnp.float32),
        pltpu.VMEM((CB * NV, W128), jnp.float32),
        pltpu.VMEM((CB * L,), jnp.float32),
        pltpu.VMEM((CB * L,), jnp.float32),
        pltpu.SemaphoreType.DMA,
    ],
)(_sc_body)


def _loss_body(pos_ref, neg_ref, out_ref):
    score = jnp.sum(pos_ref[...], axis=1)
    nscore = jnp.sum(neg_ref[...], axis=1)

    def logsig(x):
        return jnp.minimum(x, 0.0) - jnp.log1p(jnp.exp(-jnp.abs(x)))

    out_ref[0, 0] = -jnp.mean(logsig(score) + logsig(-nscore))


_loss_call = pl.pallas_call(
    _loss_body,
    out_shape=jax.ShapeDtypeStruct((1, 1), jnp.float32),
    out_specs=pl.BlockSpec(memory_space=pltpu.SMEM),
)


def kernel(u_pos, v_pos, v_neg, u_weight, v_weight):
    vidx = jnp.concatenate([v_pos[:, None], v_neg], axis=1).reshape(-1)
    uw2 = u_weight.reshape(-1, W128)
    vw2 = v_weight.reshape(-1, W128)
    pos_flat, neg_flat = _sc_call(u_pos, vidx, uw2, vw2)
    out = _loss_call(pos_flat.reshape(B, L), neg_flat.reshape(B, L))
    return out[0, 0]


# speedup vs baseline: 1.5153x; 1.5153x over previous
import functools

import jax
import jax.numpy as jnp
from jax import lax
from jax.experimental import pallas as pl
from jax.experimental.pallas import tpu as pltpu
from jax.experimental.pallas import tpu_sc as plsc

B = 16384
D = 64
NNEG = 20
NV = NNEG + 1
L = 16
NC = 2
NS = 16
NW = NC * NS
BW = B // NW
CB = 32
NCHUNK = BW // CB
GJ = 6
GN = CB * NV // GJ
W128 = 2 * D


def _sc_body(upos_hbm, vidx_hbm, uw_hbm, vw_hbm, pos_hbm, neg_hbm,
             uidx_v, urow_v, vidx_v, vrow_v, urows, vrows, posb, negb, sem):
    wid = lax.axis_index("s") * NC + lax.axis_index("c")

    def chunk_body(c, carry):
        gbase = wid * BW + c * CB

        pltpu.sync_copy(upos_hbm.at[pl.ds(gbase, CB)], uidx_v.at[pl.ds(0, CB)])
        pltpu.sync_copy(vidx_hbm.at[pl.ds(gbase * NV, CB * NV)],
                        vidx_v.at[pl.ds(0, CB * NV)])

        for i in range(CB // L):
            x = uidx_v[pl.ds(i * L, L)]
            urow_v[pl.ds(i * L, L)] = jnp.where(x >= TROWS, x - THI, x)
        for i in range(CB * NV // L):
            x = vidx_v[pl.ds(i * L, L)]
            vrow_v[pl.ds(i * L, L)] = jnp.where(x >= TROWS, x - THI, x)

        copies = [pltpu.async_copy(uw_hbm.at[urow_v], urows, sem)]
        for j in range(GJ):
            copies.append(pltpu.async_copy(
                vw_hbm.at[vrow_v.at[pl.ds(j * GN, GN)]],
                vrows.at[pl.ds(j * GN, GN)], sem))
        for cp in copies:
            cp.wait()

        def bbody(b, carry2):
            rb = b * NV

            def half_off(pe):
                return jnp.where(pe >= TROWS, D, 0)

            upar = uidx_v[pl.ds(b, L)]
            uoff = half_off(upar[0])
            u = [urows[b, pl.ds(uoff + 16 * k, 16)] for k in range(4)]
            vpar = vidx_v[pl.ds(rb, L)]
            voff = half_off(vpar[0])
            v = [vrows[rb, pl.ds(voff + 16 * k, 16)] for k in range(4)]
            aoff = half_off(vpar[1])
            acc = [vrows[rb + 1, pl.ds(aoff + 16 * k, 16)] for k in range(4)]
            vpar2 = vidx_v[pl.ds(rb + L, L)]
            for n in range(2, NV):
                pe = vpar[n] if n < L else vpar2[n - L]
                noff = half_off(pe)
                for k in range(4):
                    acc[k] = acc[k] + vrows[rb + n, pl.ds(noff + 16 * k, 16)]
            pos = u[0] * v[0] + u[1] * v[1] + u[2] * v[2] + u[3] * v[3]
            neg = u[0] * acc[0] + u[1] * acc[1] + u[2] * acc[2] + u[3] * acc[3]
            posb[pl.ds(b * L, L)] = pos
            negb[pl.ds(b * L, L)] = neg
            return carry2

        lax.fori_loop(0, CB, bbody, 0, unroll=False)

        pltpu.sync_copy(posb, pos_hbm.at[pl.ds(gbase * L, CB * L)])
        pltpu.sync_copy(negb, neg_hbm.at[pl.ds(gbase * L, CB * L)])
        return carry

    lax.fori_loop(0, NCHUNK, chunk_body, 0, unroll=False)


_sc_call = functools.partial(
    pl.kernel,
    out_type=(jax.ShapeDtypeStruct((B * L,), jnp.float32),
              jax.ShapeDtypeStruct((B * L,), jnp.float32)),
    mesh=plsc.VectorSubcoreMesh(core_axis_name="c", subcore_axis_name="s"),
    compiler_params=pltpu.CompilerParams(use_tc_tiling_on_sc=True),
    scratch_types=[
        pltpu.VMEM((CB + L,), jnp.int32),
        pltpu.VMEM((CB,), jnp.int32),
        pltpu.VMEM((CB * NV + 2 * L,), jnp.int32),
        pltpu.VMEM((CB * NV,), jnp.int32),
        pltpu.VMEM((CB, W128), jnp.float32),
        pltpu.VMEM((CB * NV, W128), jnp.float32),
        pltpu.VMEM((CB * L,), jnp.float32),
        pltpu.VMEM((CB * L,), jnp.float32),
        pltpu.SemaphoreType.DMA,
    ],
)(_sc_body)


VB = 1024
NGB = 489
THI = 488 * VB
TROWS = NGB * VB


def _tr_body(ua_ref, ub_ref, va_ref, vb_ref, uo_ref, vo_ref):
    for lo, hi, dst in ((ua_ref, ub_ref, uo_ref), (va_ref, vb_ref, vo_ref)):
        dst[...] = jnp.concatenate(
            [jnp.transpose(lo[...]), jnp.transpose(hi[...])], axis=1)


_tr_call = pl.pallas_call(
    _tr_body,
    grid=(NGB,),
    in_specs=[pl.BlockSpec((D, VB), lambda j: (0, j)),
              pl.BlockSpec((D, VB), lambda j: (0, j + 488)),
              pl.BlockSpec((D, VB), lambda j: (0, j)),
              pl.BlockSpec((D, VB), lambda j: (0, j + 488))],
    out_specs=[pl.BlockSpec((VB, W128), lambda j: (j, 0)),
               pl.BlockSpec((VB, W128), lambda j: (j, 0))],
    out_shape=[jax.ShapeDtypeStruct((TROWS, W128), jnp.float32),
               jax.ShapeDtypeStruct((TROWS, W128), jnp.float32)],
)


def _loss_body(pos_ref, neg_ref, out_ref):
    score = jnp.sum(pos_ref[...], axis=1)
    nscore = jnp.sum(neg_ref[...], axis=1)

    def logsig(x):
        return jnp.minimum(x, 0.0) - jnp.log1p(jnp.exp(-jnp.abs(x)))

    out_ref[0, 0] = -jnp.mean(logsig(score) + logsig(-nscore))


_loss_call = pl.pallas_call(
    _loss_body,
    out_shape=jax.ShapeDtypeStruct((1, 1), jnp.float32),
    out_specs=pl.BlockSpec(memory_space=pltpu.SMEM),
)


def kernel(u_pos, v_pos, v_neg, u_weight, v_weight):
    vidx = jnp.concatenate([v_pos[:, None], v_neg], axis=1).reshape(-1)
    uwT, vwT = u_weight.T, v_weight.T
    uw2, vw2 = _tr_call(uwT, uwT, vwT, vwT)
    pos_flat, neg_flat = _sc_call(u_pos, vidx, uw2, vw2)
    out = _loss_call(pos_flat.reshape(B, L), neg_flat.reshape(B, L))
    return out[0, 0]


# speedup vs baseline: 1.5856x; 1.0464x over previous
import functools

import jax
import jax.numpy as jnp
from jax import lax
from jax.experimental import pallas as pl
from jax.experimental.pallas import tpu as pltpu
from jax.experimental.pallas import tpu_sc as plsc

B = 16384
D = 64
NNEG = 20
NV = NNEG + 1
L = 16
NC = 2
NS = 16
NW = NC * NS
BW = B // NW
CB = 32
NCHUNK = BW // CB
GJ = 6
GN = CB * NV // GJ
W128 = 2 * D


def _sc_body(upos_hbm, vidx_hbm, uw_hbm, vw_hbm, pos_hbm, neg_hbm,
             uidx_v, urow_v, vidx_v, vrow_v, urows, vrows, posb, negb, sem):
    wid = lax.axis_index("s") * NC + lax.axis_index("c")

    def chunk_body(c, carry):
        gbase = wid * BW + c * CB

        pltpu.sync_copy(upos_hbm.at[pl.ds(gbase, CB)], uidx_v.at[pl.ds(0, CB)])
        pltpu.sync_copy(vidx_hbm.at[pl.ds(gbase * NV, CB * NV)],
                        vidx_v.at[pl.ds(0, CB * NV)])

        for i in range(CB // L):
            x = uidx_v[pl.ds(i * L, L)]
            urow_v[pl.ds(i * L, L)] = jnp.where(x >= TROWS, x - THI, x)
        for i in range(CB * NV // L):
            x = vidx_v[pl.ds(i * L, L)]
            vrow_v[pl.ds(i * L, L)] = jnp.where(x >= TROWS, x - THI, x)

        copies = [pltpu.async_copy(uw_hbm.at[urow_v], urows, sem)]
        for j in range(GJ):
            copies.append(pltpu.async_copy(
                vw_hbm.at[vrow_v.at[pl.ds(j * GN, GN)]],
                vrows.at[pl.ds(j * GN, GN)], sem))
        for cp in copies:
            cp.wait()

        def bbody(b, carry2):
            rb = b * NV

            def half_off(pe):
                return jnp.where(pe >= TROWS, D, 0)

            upar = uidx_v[pl.ds(b, L)]
            uoff = half_off(upar[0])
            u = [urows[b, pl.ds(uoff + 16 * k, 16)] for k in range(4)]
            vpar = vidx_v[pl.ds(rb, L)]
            voff = half_off(vpar[0])
            v = [vrows[rb, pl.ds(voff + 16 * k, 16)] for k in range(4)]
            aoff = half_off(vpar[1])
            acc = [vrows[rb + 1, pl.ds(aoff + 16 * k, 16)] for k in range(4)]
            vpar2 = vidx_v[pl.ds(rb + L, L)]
            for n in range(2, NV):
                pe = vpar[n] if n < L else vpar2[n - L]
                noff = half_off(pe)
                for k in range(4):
                    acc[k] = acc[k] + vrows[rb + n, pl.ds(noff + 16 * k, 16)]
            pos = u[0] * v[0] + u[1] * v[1] + u[2] * v[2] + u[3] * v[3]
            neg = u[0] * acc[0] + u[1] * acc[1] + u[2] * acc[2] + u[3] * acc[3]
            posb[pl.ds(b * L, L)] = pos
            negb[pl.ds(b * L, L)] = neg
            return carry2

        lax.fori_loop(0, CB, bbody, 0, unroll=False)

        pltpu.sync_copy(posb, pos_hbm.at[pl.ds(gbase * L, CB * L)])
        pltpu.sync_copy(negb, neg_hbm.at[pl.ds(gbase * L, CB * L)])
        return carry

    lax.fori_loop(0, NCHUNK, chunk_body, 0, unroll=False)


_sc_call = functools.partial(
    pl.kernel,
    out_type=(jax.ShapeDtypeStruct((B * L,), jnp.float32),
              jax.ShapeDtypeStruct((B * L,), jnp.float32)),
    mesh=plsc.VectorSubcoreMesh(core_axis_name="c", subcore_axis_name="s"),
    compiler_params=pltpu.CompilerParams(use_tc_tiling_on_sc=True),
    scratch_types=[
        pltpu.VMEM((CB + L,), jnp.int32),
        pltpu.VMEM((CB,), jnp.int32),
        pltpu.VMEM((CB * NV + 2 * L,), jnp.int32),
        pltpu.VMEM((CB * NV,), jnp.int32),
        pltpu.VMEM((CB, W128), jnp.float32),
        pltpu.VMEM((CB * NV, W128), jnp.float32),
        pltpu.VMEM((CB * L,), jnp.float32),
        pltpu.VMEM((CB * L,), jnp.float32),
        pltpu.SemaphoreType.DMA,
    ],
)(_sc_body)


VB = 1024
NGB = 489
THI = 488 * VB
TROWS = NGB * VB


def _tr_body(ua_ref, ub_ref, va_ref, vb_ref, uo_ref, vo_ref):
    ii = lax.broadcasted_iota(jnp.int32, (D, W128), 0)
    jj = lax.broadcasted_iota(jnp.int32, (D, W128), 1)
    sel_lo = (ii == jj).astype(jnp.float32)
    sel_hi = (ii == jj - D).astype(jnp.float32)
    dims = (((0,), (0,)), ((), ()))
    for lo, hi, dst in ((ua_ref, ub_ref, uo_ref), (va_ref, vb_ref, vo_ref)):
        dst[...] = (
            lax.dot_general(lo[...], sel_lo, dims,
                            preferred_element_type=jnp.float32)
            + lax.dot_general(hi[...], sel_hi, dims,
                              preferred_element_type=jnp.float32))


_tr_call = pl.pallas_call(
    _tr_body,
    grid=(NGB,),
    in_specs=[pl.BlockSpec((D, VB), lambda j: (0, j)),
              pl.BlockSpec((D, VB), lambda j: (0, j + 488)),
              pl.BlockSpec((D, VB), lambda j: (0, j)),
              pl.BlockSpec((D, VB), lambda j: (0, j + 488))],
    out_specs=[pl.BlockSpec((VB, W128), lambda j: (j, 0)),
               pl.BlockSpec((VB, W128), lambda j: (j, 0))],
    out_shape=[jax.ShapeDtypeStruct((TROWS, W128), jnp.float32),
               jax.ShapeDtypeStruct((TROWS, W128), jnp.float32)],
)


def _loss_body(pos_ref, neg_ref, out_ref):
    score = jnp.sum(pos_ref[...], axis=1)
    nscore = jnp.sum(neg_ref[...], axis=1)

    def logsig(x):
        return jnp.minimum(x, 0.0) - jnp.log1p(jnp.exp(-jnp.abs(x)))

    out_ref[0, 0] = -jnp.mean(logsig(score) + logsig(-nscore))


_loss_call = pl.pallas_call(
    _loss_body,
    out_shape=jax.ShapeDtypeStruct((1, 1), jnp.float32),
    out_specs=pl.BlockSpec(memory_space=pltpu.SMEM),
)


def kernel(u_pos, v_pos, v_neg, u_weight, v_weight):
    vidx = jnp.concatenate([v_pos[:, None], v_neg], axis=1).reshape(-1)
    uwT, vwT = u_weight.T, v_weight.T
    uw2, vw2 = _tr_call(uwT, uwT, vwT, vwT)
    pos_flat, neg_flat = _sc_call(u_pos, vidx, uw2, vw2)
    out = _loss_call(pos_flat.reshape(B, L), neg_flat.reshape(B, L))
    return out[0, 0]


# speedup vs baseline: 1.5940x; 1.0053x over previous
import functools

import jax
import jax.numpy as jnp
from jax import lax
from jax.experimental import pallas as pl
from jax.experimental.pallas import tpu as pltpu
from jax.experimental.pallas import tpu_sc as plsc

B = 16384
D = 64
NNEG = 20
NV = NNEG + 1
L = 16
NC = 2
NS = 16
NW = NC * NS
BW = B // NW
CB = 32
NCHUNK = BW // CB
GJ = 6
GN = CB * NV // GJ
W128 = 2 * D


def _sc_body(upos_hbm, vidx_hbm, uw_hbm, vw_hbm, pos_hbm, neg_hbm,
             uidx_v, urow_v, vidx_v, vrow_v, urows, vrows, posb, negb, sem):
    wid = lax.axis_index("s") * NC + lax.axis_index("c")

    def chunk_body(c, carry):
        gbase = wid * BW + c * CB

        pltpu.sync_copy(upos_hbm.at[pl.ds(gbase, CB)], uidx_v.at[pl.ds(0, CB)])
        pltpu.sync_copy(vidx_hbm.at[pl.ds(gbase * NV, CB * NV)],
                        vidx_v.at[pl.ds(0, CB * NV)])

        def to_row(x):
            return lax.shift_left(lax.shift_right_logical(x, 11), 10) | (
                x & 1023)

        for i in range(CB // L):
            x = uidx_v[pl.ds(i * L, L)]
            urow_v[pl.ds(i * L, L)] = to_row(x)
        for i in range(CB * NV // L):
            x = vidx_v[pl.ds(i * L, L)]
            vrow_v[pl.ds(i * L, L)] = to_row(x)

        copies = [pltpu.async_copy(uw_hbm.at[urow_v], urows, sem)]
        for j in range(GJ):
            copies.append(pltpu.async_copy(
                vw_hbm.at[vrow_v.at[pl.ds(j * GN, GN)]],
                vrows.at[pl.ds(j * GN, GN)], sem))
        for cp in copies:
            cp.wait()

        def bbody(b, carry2):
            rb = b * NV

            def half_off(pe):
                return (lax.shift_right_logical(pe, 10) & 1) * D

            upar = uidx_v[pl.ds(b, L)]
            uoff = half_off(upar[0])
            u = [urows[b, pl.ds(uoff + 16 * k, 16)] for k in range(4)]
            vpar = vidx_v[pl.ds(rb, L)]
            voff = half_off(vpar[0])
            v = [vrows[rb, pl.ds(voff + 16 * k, 16)] for k in range(4)]
            aoff = half_off(vpar[1])
            acc = [vrows[rb + 1, pl.ds(aoff + 16 * k, 16)] for k in range(4)]
            vpar2 = vidx_v[pl.ds(rb + L, L)]
            for n in range(2, NV):
                pe = vpar[n] if n < L else vpar2[n - L]
                noff = half_off(pe)
                for k in range(4):
                    acc[k] = acc[k] + vrows[rb + n, pl.ds(noff + 16 * k, 16)]
            pos = u[0] * v[0] + u[1] * v[1] + u[2] * v[2] + u[3] * v[3]
            neg = u[0] * acc[0] + u[1] * acc[1] + u[2] * acc[2] + u[3] * acc[3]
            posb[pl.ds(b * L, L)] = pos
            negb[pl.ds(b * L, L)] = neg
            return carry2

        lax.fori_loop(0, CB, bbody, 0, unroll=False)

        pltpu.sync_copy(posb, pos_hbm.at[pl.ds(gbase * L, CB * L)])
        pltpu.sync_copy(negb, neg_hbm.at[pl.ds(gbase * L, CB * L)])
        return carry

    lax.fori_loop(0, NCHUNK, chunk_body, 0, unroll=False)


_sc_call = functools.partial(
    pl.kernel,
    out_type=(jax.ShapeDtypeStruct((B * L,), jnp.float32),
              jax.ShapeDtypeStruct((B * L,), jnp.float32)),
    mesh=plsc.VectorSubcoreMesh(core_axis_name="c", subcore_axis_name="s"),
    compiler_params=pltpu.CompilerParams(use_tc_tiling_on_sc=True),
    scratch_types=[
        pltpu.VMEM((CB + L,), jnp.int32),
        pltpu.VMEM((CB,), jnp.int32),
        pltpu.VMEM((CB * NV + 2 * L,), jnp.int32),
        pltpu.VMEM((CB * NV,), jnp.int32),
        pltpu.VMEM((CB, W128), jnp.float32),
        pltpu.VMEM((CB * NV, W128), jnp.float32),
        pltpu.VMEM((CB * L,), jnp.float32),
        pltpu.VMEM((CB * L,), jnp.float32),
        pltpu.SemaphoreType.DMA,
    ],
)(_sc_body)


VB = 1024
NGB = 489
TROWS = NGB * VB


def _tr_body(ua_ref, ub_ref, va_ref, vb_ref, uo_ref, vo_ref):
    ii = lax.broadcasted_iota(jnp.int32, (D, W128), 0)
    jj = lax.broadcasted_iota(jnp.int32, (D, W128), 1)
    sel_lo = (ii == jj).astype(jnp.float32)
    sel_hi = (ii == jj - D).astype(jnp.float32)
    dims = (((0,), (0,)), ((), ()))
    for lo, hi, dst in ((ua_ref, ub_ref, uo_ref), (va_ref, vb_ref, vo_ref)):
        dst[...] = (
            lax.dot_general(lo[...], sel_lo, dims,
                            preferred_element_type=jnp.float32)
            + lax.dot_general(hi[...], sel_hi, dims,
                              preferred_element_type=jnp.float32))


_tr_call = pl.pallas_call(
    _tr_body,
    grid=(NGB,),
    in_specs=[pl.BlockSpec((D, VB), lambda j: (0, 2 * j)),
              pl.BlockSpec((D, VB), lambda j: (0, jnp.minimum(2 * j + 1, 976))),
              pl.BlockSpec((D, VB), lambda j: (0, 2 * j)),
              pl.BlockSpec((D, VB), lambda j: (0, jnp.minimum(2 * j + 1, 976)))],
    out_specs=[pl.BlockSpec((VB, W128), lambda j: (j, 0)),
               pl.BlockSpec((VB, W128), lambda j: (j, 0))],
    out_shape=[jax.ShapeDtypeStruct((TROWS, W128), jnp.float32),
               jax.ShapeDtypeStruct((TROWS, W128), jnp.float32)],
)


def _loss_body(pos_ref, neg_ref, out_ref):
    score = jnp.sum(pos_ref[...], axis=1)
    nscore = jnp.sum(neg_ref[...], axis=1)

    def logsig(x):
        return jnp.minimum(x, 0.0) - jnp.log1p(jnp.exp(-jnp.abs(x)))

    out_ref[0, 0] = -jnp.mean(logsig(score) + logsig(-nscore))


_loss_call = pl.pallas_call(
    _loss_body,
    out_shape=jax.ShapeDtypeStruct((1, 1), jnp.float32),
    out_specs=pl.BlockSpec(memory_space=pltpu.SMEM),
)


def kernel(u_pos, v_pos, v_neg, u_weight, v_weight):
    vidx = jnp.concatenate([v_pos[:, None], v_neg], axis=1).reshape(-1)
    uwT, vwT = u_weight.T, v_weight.T
    uw2, vw2 = _tr_call(uwT, uwT, vwT, vwT)
    pos_flat, neg_flat = _sc_call(u_pos, vidx, uw2, vw2)
    out = _loss_call(pos_flat.reshape(B, L), neg_flat.reshape(B, L))
    return out[0, 0]
